# baseline (speedup 1.0000x reference)
import functools

import jax
import jax.numpy as jnp
from jax import lax
from jax.experimental import pallas as pl
from jax.experimental.pallas import tpu as pltpu
from jax.experimental.pallas import tpu_sc as plsc

NBINS = 4096
C = 21
NW = 32
HSIZE = C * NBINS


def _sc_hist_kernel(pred_hbm, tgt_hbm, out_hbm, hist, pv, tv):
    wid = lax.axis_index("s") * 2 + lax.axis_index("c")
    n = wid // 8
    s0 = (wid % 8) * 32768

    zeros16 = jnp.zeros((16,), jnp.int32)

    @plsc.parallel_loop(0, HSIZE // 64, unroll=4)
    def zero_body(i):
        hist[pl.ds(i * 64, 16)] = zeros16
        hist[pl.ds(i * 64 + 16, 16)] = zeros16
        hist[pl.ds(i * 64 + 32, 16)] = zeros16
        hist[pl.ds(i * 64 + 48, 16)] = zeros16

    lanes = lax.iota(jnp.int32, 16)
    ones = jnp.ones((16,), jnp.int32)
    negones = -ones
    fgval = jnp.full((16,), 65536, jnp.int32)

    def chunk_body(ci, _):
        base = s0 + ci * 1024
        pltpu.sync_copy(pred_hbm.at[n, :, pl.ds(base, 1024)], pv)
        pltpu.sync_copy(tgt_hbm.at[n, pl.ds(base, 1024)], tv)

        @plsc.parallel_loop(0, 64, unroll=4)
        def vec_body(v):
            for l in range(C):
                p = pv[l, pl.ds(v * 16, 16)]
                binp = jnp.minimum((p * float(NBINS)).astype(jnp.int32),
                                   NBINS - 1)
                plsc.addupdate_scatter(hist, [binp + l * NBINS], ones)
            t = tv[pl.ds(v * 16, 16)]
            pt = plsc.load_gather(pv, [t, lanes + v * 16])
            bint = jnp.minimum((pt * float(NBINS)).astype(jnp.int32),
                               NBINS - 1)
            tbase = t * NBINS
            plsc.addupdate_scatter(hist, [tbase + bint], negones)
            plsc.addupdate_scatter(hist, [tbase + (NBINS - 1) - bint], fgval)

        return 0

    lax.fori_loop(0, 32, chunk_body, 0)
    pltpu.sync_copy(hist, out_hbm.at[wid])


def _make_sc_hist():
    mesh = plsc.VectorSubcoreMesh(core_axis_name="c", subcore_axis_name="s")
    return pl.kernel(
        _sc_hist_kernel,
        out_type=jax.ShapeDtypeStruct((NW, HSIZE), jnp.int32),
        mesh=mesh,
        scratch_types=[
            pltpu.VMEM((HSIZE,), jnp.int32),
            pltpu.VMEM((C, 1024), jnp.float32),
            pltpu.VMEM((1024,), jnp.int32),
        ],
        compiler_params=pltpu.CompilerParams(needs_layout_passes=False),
    )


def _shift_down(x, s):
    r, nb = x.shape
    return jnp.concatenate(
        [x[:, s:], jnp.zeros((r, s), x.dtype)], axis=1)


def _tc_loss_kernel(hist_ref, out_ref):
    h = hist_ref[...]
    fgh_i = lax.shift_right_logical(h, 16)
    bgh_i = jnp.bitwise_and(h, 0xFFFF)
    fgh = jnp.sum(fgh_i, axis=0).astype(jnp.float32)
    bgh = jnp.sum(bgh_i, axis=0).astype(jnp.float32)

    G = jnp.sum(fgh, axis=1, keepdims=True)

    sf = fgh
    sb = bgh
    s = 1
    while s < NBINS:
        sf = sf + _shift_down(sf, s)
        sb = sb + _shift_down(sb, s)
        s *= 2
    Fm = sf - 0.5 * fgh
    Bm = sb - 0.5 * bgh
    denom = jnp.maximum(G + Bm, 0.5)
    J = 1.0 - (G - Fm) / denom
    lossv = jnp.sum(J, axis=1) / float(NBINS)
    pres = (G[:, 0] > 0.0).astype(jnp.float32)
    out = jnp.sum(lossv * pres) / jnp.maximum(jnp.sum(pres), 1.0)
    out_ref[...] = jnp.broadcast_to(out, (1, 1))


@jax.jit
def kernel(predict, target):
    n, c, h, w = predict.shape
    pred = predict.reshape(n, c, h * w)
    tgt = target.reshape(n, h * w)

    hist = _make_sc_hist()(pred, tgt)
    hist3 = hist.reshape(NW, C, NBINS)

    out = pl.pallas_call(
        _tc_loss_kernel,
        out_shape=jax.ShapeDtypeStruct((1, 1), jnp.float32),
    )(hist3)
    return out.reshape(())


# speedup vs baseline: 1.0114x; 1.0114x over previous
import functools

import jax
import jax.numpy as jnp
from jax import lax
from jax.experimental import pallas as pl
from jax.experimental.pallas import tpu as pltpu
from jax.experimental.pallas import tpu_sc as plsc

NBINS = 4096
C = 21
NW = 32
HSIZE = C * NBINS


def _sc_hist_kernel(pred_hbm, tgt_hbm, out_hbm, hist, pv, tv, sp0, sp1,
                    st0, st1):
    wid = lax.axis_index("s") * 2 + lax.axis_index("c")
    n = wid // 8
    s0 = (wid % 8) * 32768
    psems = [sp0, sp1]
    tsems = [st0, st1]

    zeros16 = jnp.zeros((16,), jnp.int32)

    @plsc.parallel_loop(0, HSIZE // 64, unroll=4)
    def zero_body(i):
        hist[pl.ds(i * 64, 16)] = zeros16
        hist[pl.ds(i * 64 + 16, 16)] = zeros16
        hist[pl.ds(i * 64 + 32, 16)] = zeros16
        hist[pl.ds(i * 64 + 48, 16)] = zeros16

    lanes = lax.iota(jnp.int32, 16)
    ones = jnp.ones((16,), jnp.int32)
    negones = -ones
    fgval = jnp.full((16,), 65536, jnp.int32)

    CH = 512
    NCH = 32768 // CH

    def start(ci, b):
        pltpu.async_copy(pred_hbm.at[n, :, pl.ds(s0 + ci * CH, CH)],
                         pv.at[b], psems[b])
        pltpu.async_copy(tgt_hbm.at[n, pl.ds(s0 + ci * CH, CH)],
                         tv.at[b], tsems[b])

    start(0, 0)
    start(1, 1)

    def pair_body(gi, _):
        for b in range(2):
            ci = gi * 2 + b
            pltpu.make_async_copy(pred_hbm.at[n, :, pl.ds(s0, CH)],
                                  pv.at[b], psems[b]).wait()
            pltpu.make_async_copy(tgt_hbm.at[n, pl.ds(s0, CH)],
                                  tv.at[b], tsems[b]).wait()

            @plsc.parallel_loop(0, CH // 16, unroll=4)
            def vec_body(v):
                for l in range(C):
                    p = pv[b, l, pl.ds(v * 16, 16)]
                    binp = jnp.minimum((p * float(NBINS)).astype(jnp.int32),
                                       NBINS - 1)
                    plsc.addupdate_scatter(hist, [binp + l * NBINS], ones)
                t = tv[b, pl.ds(v * 16, 16)]
                pt = plsc.load_gather(pv.at[b], [t, lanes + v * 16])
                bint = jnp.minimum((pt * float(NBINS)).astype(jnp.int32),
                                   NBINS - 1)
                tbase = t * NBINS
                plsc.addupdate_scatter(hist, [tbase + bint], negones)
                plsc.addupdate_scatter(hist, [tbase + (NBINS - 1) - bint],
                                       fgval)

            @pl.when(ci + 2 < NCH)
            def _():
                start(ci + 2, b)

        return 0

    lax.fori_loop(0, NCH // 2, pair_body, 0)
    pltpu.sync_copy(hist, out_hbm.at[wid])


def _make_sc_hist():
    mesh = plsc.VectorSubcoreMesh(core_axis_name="c", subcore_axis_name="s")
    return pl.kernel(
        _sc_hist_kernel,
        out_type=jax.ShapeDtypeStruct((NW, HSIZE), jnp.int32),
        mesh=mesh,
        scratch_types=[
            pltpu.VMEM((HSIZE,), jnp.int32),
            pltpu.VMEM((2, C, 512), jnp.float32),
            pltpu.VMEM((2, 512), jnp.int32),
            pltpu.SemaphoreType.DMA,
            pltpu.SemaphoreType.DMA,
            pltpu.SemaphoreType.DMA,
            pltpu.SemaphoreType.DMA,
        ],
        compiler_params=pltpu.CompilerParams(needs_layout_passes=False),
    )


def _shift_down(x, s):
    r, nb = x.shape
    return jnp.concatenate(
        [x[:, s:], jnp.zeros((r, s), x.dtype)], axis=1)


def _tc_loss_kernel(hist_ref, out_ref):
    h = hist_ref[...]
    fgh_i = lax.shift_right_logical(h, 16)
    bgh_i = jnp.bitwise_and(h, 0xFFFF)
    fgh = jnp.sum(fgh_i, axis=0).astype(jnp.float32)
    bgh = jnp.sum(bgh_i, axis=0).astype(jnp.float32)

    G = jnp.sum(fgh, axis=1, keepdims=True)

    sf = fgh
    sb = bgh
    s = 1
    while s < NBINS:
        sf = sf + _shift_down(sf, s)
        sb = sb + _shift_down(sb, s)
        s *= 2
    Fm = sf - 0.5 * fgh
    Bm = sb - 0.5 * bgh
    denom = jnp.maximum(G + Bm, 0.5)
    J = 1.0 - (G - Fm) / denom
    lossv = jnp.sum(J, axis=1) / float(NBINS)
    pres = (G[:, 0] > 0.0).astype(jnp.float32)
    out = jnp.sum(lossv * pres) / jnp.maximum(jnp.sum(pres), 1.0)
    out_ref[...] = jnp.broadcast_to(out, (1, 1))


@jax.jit
def kernel(predict, target):
    n, c, h, w = predict.shape
    pred = predict.reshape(n, c, h * w)
    tgt = target.reshape(n, h * w)

    hist = _make_sc_hist()(pred, tgt)
    hist3 = hist.reshape(NW, C, NBINS)

    out = pl.pallas_call(
        _tc_loss_kernel,
        out_shape=jax.ShapeDtypeStruct((1, 1), jnp.float32),
    )(hist3)
    return out.reshape(())


# speedup vs baseline: 1.5609x; 1.5432x over previous
import functools

import jax
import jax.numpy as jnp
from jax import lax
from jax.experimental import pallas as pl
from jax.experimental.pallas import tpu as pltpu
from jax.experimental.pallas import tpu_sc as plsc

NBINS = 4096
C = 21
NW = 32
HSIZE = C * NBINS


def _sc_hist_kernel(pred_hbm, tgt_hbm, out_hbm, hist, pv, tv, sp0, sp1,
                    st0, st1):
    wid = lax.axis_index("s") * 2 + lax.axis_index("c")
    n = wid // 8
    r0 = (wid % 8) * 64
    psems = [sp0, sp1]
    tsems = [st0, st1]

    zeros16 = jnp.zeros((16,), jnp.int32)

    @plsc.parallel_loop(0, HSIZE // 64, unroll=4)
    def zero_body(i):
        hist[pl.ds(i * 64, 16)] = zeros16
        hist[pl.ds(i * 64 + 16, 16)] = zeros16
        hist[pl.ds(i * 64 + 32, 16)] = zeros16
        hist[pl.ds(i * 64 + 48, 16)] = zeros16

    lanes = lax.iota(jnp.int32, 16)
    ones = jnp.ones((16,), jnp.int32)
    negones = -ones
    fgval = jnp.full((16,), 65536, jnp.int32)

    CH = 512
    NCH = 64

    def start(ci, b):
        pltpu.async_copy(pred_hbm.at[n, :, r0 + ci, :], pv.at[b], psems[b])
        pltpu.async_copy(tgt_hbm.at[n, r0 + ci, :], tv.at[b], tsems[b])

    start(0, 0)
    start(1, 1)

    def pair_body(gi, _):
        for b in range(2):
            ci = gi * 2 + b
            pltpu.make_async_copy(pred_hbm.at[n, :, r0, :],
                                  pv.at[b], psems[b]).wait()
            pltpu.make_async_copy(tgt_hbm.at[n, r0, :],
                                  tv.at[b], tsems[b]).wait()

            @plsc.parallel_loop(0, CH // 16, unroll=4)
            def vec_body(v):
                for l in range(C):
                    p = pv[b, l, pl.ds(v * 16, 16)]
                    binp = jnp.minimum((p * float(NBINS)).astype(jnp.int32),
                                       NBINS - 1)
                    plsc.addupdate_scatter(hist, [binp + l * NBINS], ones)
                t = tv[b, pl.ds(v * 16, 16)]
                pt = plsc.load_gather(pv.at[b], [t, lanes + v * 16])
                bint = jnp.minimum((pt * float(NBINS)).astype(jnp.int32),
                                   NBINS - 1)
                tbase = t * NBINS
                plsc.addupdate_scatter(hist, [tbase + bint], negones)
                plsc.addupdate_scatter(hist, [tbase + (NBINS - 1) - bint],
                                       fgval)

            @pl.when(ci + 2 < NCH)
            def _():
                start(ci + 2, b)

        return 0

    lax.fori_loop(0, NCH // 2, pair_body, 0)
    pltpu.sync_copy(hist, out_hbm.at[wid])


def _make_sc_hist():
    mesh = plsc.VectorSubcoreMesh(core_axis_name="c", subcore_axis_name="s")
    return pl.kernel(
        _sc_hist_kernel,
        out_type=jax.ShapeDtypeStruct((NW, HSIZE), jnp.int32),
        mesh=mesh,
        scratch_types=[
            pltpu.VMEM((HSIZE,), jnp.int32),
            pltpu.VMEM((2, C, 512), jnp.float32),
            pltpu.VMEM((2, 512), jnp.int32),
            pltpu.SemaphoreType.DMA,
            pltpu.SemaphoreType.DMA,
            pltpu.SemaphoreType.DMA,
            pltpu.SemaphoreType.DMA,
        ],
        compiler_params=pltpu.CompilerParams(needs_layout_passes=False),
    )


def _shift_down(x, s):
    r, nb = x.shape
    return jnp.concatenate(
        [x[:, s:], jnp.zeros((r, s), x.dtype)], axis=1)


def _tc_loss_kernel(hist_ref, out_ref):
    h = hist_ref[...]
    fgh_i = lax.shift_right_logical(h, 16)
    bgh_i = jnp.bitwise_and(h, 0xFFFF)
    fgh = jnp.sum(fgh_i, axis=0).astype(jnp.float32)
    bgh = jnp.sum(bgh_i, axis=0).astype(jnp.float32)

    G = jnp.sum(fgh, axis=1, keepdims=True)

    sf = fgh
    sb = bgh
    s = 1
    while s < NBINS:
        sf = sf + _shift_down(sf, s)
        sb = sb + _shift_down(sb, s)
        s *= 2
    Fm = sf - 0.5 * fgh
    Bm = sb - 0.5 * bgh
    denom = jnp.maximum(G + Bm, 0.5)
    J = 1.0 - (G - Fm) / denom
    lossv = jnp.sum(J, axis=1) / float(NBINS)
    pres = (G[:, 0] > 0.0).astype(jnp.float32)
    out = jnp.sum(lossv * pres) / jnp.maximum(jnp.sum(pres), 1.0)
    out_ref[...] = jnp.broadcast_to(out, (1, 1))


@jax.jit
def kernel(predict, target):
    hist = _make_sc_hist()(predict, target)
    hist3 = hist.reshape(NW, C, NBINS)

    out = pl.pallas_call(
        _tc_loss_kernel,
        out_shape=jax.ShapeDtypeStruct((1, 1), jnp.float32),
    )(hist3)
    return out.reshape(())
